# baseline (speedup 1.0000x reference)
import functools

import jax
import jax.numpy as jnp
from jax import lax
from jax.experimental import pallas as pl
from jax.experimental.pallas import tpu as pltpu
from jax.experimental.pallas import tpu_sc as plsc

N = 10000
D = 128
DH = D // 2
E = 320000
NC, NS = 2, 16
CH = 128
R = 10240
EPW = 20480
NCHUNK = EPW // CH
EP = NS * EPW
RPT = R // NS
CNTW = 16
GB = 40
NGB = NCHUNK // GB
NBUF = 8


def _mm_body(x_ref, w_ref, o0_ref, o1_ref):
    b = jnp.dot(x_ref[...], w_ref[...], preferred_element_type=jnp.float32)
    o0_ref[...] = b[:, :DH]
    o1_ref[...] = b[:, DH:]


def _combine_body(x_ref, wc_ref, b_ref, s0_ref, s1_ref, c_ref, o_ref):
    alpha = jnp.dot(x_ref[...], wc_ref[...],
                    preferred_element_type=jnp.float32)
    s = jnp.concatenate([s0_ref[0:N, :], s1_ref[0:N, :]], axis=1)
    c = c_ref[0:N, 0:1]
    gamma = s / jnp.maximum(c, 1.0)
    o_ref[...] = alpha + gamma + b_ref[...]


def _sc_body(beta0, beta1, dsti, srci, s0_o, s1_o, cnt_o,
             dstv, srcv, rows, rows1, rows2, rows3, rows4, rows5, rows6,
             rows7, ones_v, acc_s, acc_c,
             gsa, gsb, gsc, gsd, gse, gsf, gsg, gsh,
             ssa, ssb, ssc, ssd, sse, ssf, ssg, ssh, csem):
    cid = lax.axis_index("c")
    sid = lax.axis_index("s")
    z16 = jnp.zeros((16,), jnp.float32)
    o16 = jnp.ones((16,), jnp.float32)

    @pl.loop(0, CH)
    def _(j):
        for k in range(DH // 16):
            rows[j, pl.ds(k * 16, 16)] = z16
        ones_v[j, pl.ds(0, 16)] = z16

    @pl.loop(0, RPT // CH)
    def _(k):
        base = sid * RPT + k * CH
        pltpu.sync_copy(rows, acc_s.at[pl.ds(base, CH)])
        pltpu.sync_copy(ones_v, acc_c.at[pl.ds(base, CH)])

    @pl.loop(0, CH)
    def _(j):
        ones_v[j, pl.ds(0, 16)] = o16

    plsc.subcore_barrier()

    def run(beta_h, with_cnt):
        bufs = (rows, rows1, rows2, rows3, rows4, rows5, rows6, rows7)
        gsems = (gsa, gsb, gsc, gsd, gse, gsf, gsg, gsh)
        ssems = (ssa, ssb, ssc, ssd, sse, ssf, ssg, ssh)

        def g_start(buf, j):
            pltpu.async_copy(beta_h.at[dstv.at[j]], bufs[buf], gsems[buf])

        def g_wait(buf, j):
            pltpu.make_async_copy(
                beta_h.at[dstv.at[j]], bufs[buf], gsems[buf]).wait()

        def s_start(buf, j):
            pltpu.async_copy(bufs[buf], acc_s.at[srcv.at[j]], ssems[buf],
                             add=True)
            if with_cnt:
                pltpu.async_copy(ones_v, acc_c.at[srcv.at[j]], csem,
                                 add=True)

        def s_wait(buf, j):
            pltpu.make_async_copy(
                bufs[buf], acc_s.at[srcv.at[j]], ssems[buf]).wait()
            if with_cnt:
                pltpu.make_async_copy(
                    ones_v, acc_c.at[srcv.at[j]], csem).wait()

        @pl.loop(0, NGB)
        def _(g):
            pltpu.sync_copy(dsti.at[pl.ds(sid * NCHUNK + g * GB, GB)], dstv)
            pltpu.sync_copy(srci.at[pl.ds(sid * NCHUNK + g * GB, GB)], srcv)
            hd = NBUF // 2
            for t in range(hd):
                g_start(t, t)

            @pl.loop(0, GB, step=NBUF)
            def _(j):
                for t in range(NBUF):
                    b, bp = t, (t + hd) % NBUF
                    c = j + t
                    g_wait(b, c)
                    s_start(b, c)

                    @pl.when(c >= hd)
                    def _():
                        s_wait(bp, c - hd)

                    @pl.when(c + hd < GB)
                    def _():
                        g_start(bp, c + hd)

            for t in range(hd):
                s_wait(t + hd, GB - hd + t)

    @pl.when(cid == 0)
    def _():
        run(beta0, True)

    @pl.when(cid == 1)
    def _():
        run(beta1, False)

    plsc.subcore_barrier()

    @pl.loop(0, RPT // CH)
    def _(k):
        base = sid * RPT + k * CH
        pltpu.sync_copy(acc_s.at[pl.ds(base, CH)], rows)

        @pl.when(cid == 0)
        def _():
            pltpu.sync_copy(rows, s0_o.at[pl.ds(base, CH)])
            pltpu.sync_copy(acc_c.at[pl.ds(base, CH)], ones_v)
            pltpu.sync_copy(ones_v, cnt_o.at[pl.ds(base, CH)])

        @pl.when(cid == 1)
        def _():
            pltpu.sync_copy(rows, s1_o.at[pl.ds(base, CH)])


_sc_aggregate = functools.partial(
    pl.kernel,
    out_type=[
        jax.ShapeDtypeStruct((R, DH), jnp.float32),
        jax.ShapeDtypeStruct((R, DH), jnp.float32),
        jax.ShapeDtypeStruct((R, CNTW), jnp.float32),
    ],
    mesh=plsc.VectorSubcoreMesh(core_axis_name="c", subcore_axis_name="s"),
    compiler_params=pltpu.CompilerParams(use_tc_tiling_on_sc=False),
    scratch_types=[
        pltpu.VMEM((GB, CH), jnp.int32),
        pltpu.VMEM((GB, CH), jnp.int32),
    ] + [pltpu.VMEM((CH, DH), jnp.float32)] * NBUF
      + [
        pltpu.VMEM((CH, CNTW), jnp.float32),
        pltpu.VMEM_SHARED((R, DH), jnp.float32),
        pltpu.VMEM_SHARED((R, CNTW), jnp.float32),
    ] + [pltpu.SemaphoreType.DMA] * (2 * NBUF + 1),
)(_sc_body)


def kernel(x, edge_index, wc, wn, bias):
    src = edge_index[0].astype(jnp.int32)
    dst = edge_index[1].astype(jnp.int32)
    pad = EP - E
    pad_iota = jnp.arange(pad, dtype=jnp.int32)
    src_p = jnp.concatenate(
        [src, N + pad_iota % (R - N)]).reshape(EP // CH, CH)
    dst_p = jnp.concatenate(
        [dst, pad_iota % N]).reshape(EP // CH, CH)

    beta0, beta1 = pl.pallas_call(
        _mm_body,
        out_shape=[
            jax.ShapeDtypeStruct((N, DH), jnp.float32),
            jax.ShapeDtypeStruct((N, DH), jnp.float32),
        ],
    )(x, wn)

    s0, s1, cnt = _sc_aggregate(beta0, beta1, dst_p, src_p)

    out = pl.pallas_call(
        _combine_body,
        out_shape=jax.ShapeDtypeStruct((N, D), jnp.float32),
    )(x, wc, bias.reshape(1, D), s0, s1, cnt)
    return out


# speedup vs baseline: 1.0101x; 1.0101x over previous
import functools

import jax
import jax.numpy as jnp
from jax import lax
from jax.experimental import pallas as pl
from jax.experimental.pallas import tpu as pltpu
from jax.experimental.pallas import tpu_sc as plsc

N = 10000
D = 128
DH = D // 2
E = 320000
NC, NS = 2, 16
CH = 128
R = 10240
EPW = 20480
NCHUNK = EPW // CH
EP = NS * EPW
RPT = R // NS
CNTW = 16
GB = 40
NGB = NCHUNK // GB
NBUF = 8


def _mm_body(x_ref, w_ref, o0_ref, o1_ref):
    b = jnp.dot(x_ref[...], w_ref[...], preferred_element_type=jnp.float32)
    o0_ref[...] = b[:, :DH]
    o1_ref[...] = b[:, DH:]


def _combine_body(x_ref, wc_ref, b_ref, s0_ref, s1_ref, c_ref, o_ref):
    alpha = jnp.dot(x_ref[...], wc_ref[...],
                    preferred_element_type=jnp.float32)
    s = jnp.concatenate([s0_ref[0:N, :], s1_ref[0:N, :]], axis=1)
    c = c_ref[0:N, 0:1]
    gamma = s / jnp.maximum(c, 1.0)
    o_ref[...] = alpha + gamma + b_ref[...]


def _sc_body(beta0, beta1, dsti, srci, s0_o, s1_o, cnt_o,
             dstv, srcv, rows, rows1, rows2, rows3, rows4, rows5, rows6,
             rows7, ones_v, acc_s, acc_c,
             gsa, gsb, gsc, gsd, gse, gsf, gsg, gsh,
             ssa, ssb, ssc, ssd, sse, ssf, ssg, ssh, csem):
    cid = lax.axis_index("c")
    sid = lax.axis_index("s")
    z16 = jnp.zeros((16,), jnp.float32)
    o16 = jnp.ones((16,), jnp.float32)

    @pl.loop(0, CH)
    def _(j):
        for k in range(DH // 16):
            rows[j, pl.ds(k * 16, 16)] = z16
        ones_v[j, pl.ds(0, 16)] = z16

    @pl.loop(0, RPT // CH)
    def _(k):
        base = sid * RPT + k * CH
        pltpu.async_copy(rows, acc_s.at[pl.ds(base, CH)], gsa)

        @pl.when(cid == 0)
        def _():
            pltpu.async_copy(ones_v, acc_c.at[pl.ds(base, CH)], csem)

    @pl.loop(0, RPT // CH)
    def _(k):
        base = sid * RPT + k * CH
        pltpu.make_async_copy(rows, acc_s.at[pl.ds(base, CH)], gsa).wait()

        @pl.when(cid == 0)
        def _():
            pltpu.make_async_copy(
                ones_v, acc_c.at[pl.ds(base, CH)], csem).wait()

    @pl.loop(0, CH)
    def _(j):
        ones_v[j, pl.ds(0, 16)] = o16

    plsc.subcore_barrier()

    def run(beta_h, with_cnt):
        bufs = (rows, rows1, rows2, rows3, rows4, rows5, rows6, rows7)
        gsems = (gsa, gsb, gsc, gsd, gse, gsf, gsg, gsh)
        ssems = (ssa, ssb, ssc, ssd, sse, ssf, ssg, ssh)

        def g_start(buf, j):
            pltpu.async_copy(beta_h.at[dstv.at[j]], bufs[buf], gsems[buf])

        def g_wait(buf, j):
            pltpu.make_async_copy(
                beta_h.at[dstv.at[j]], bufs[buf], gsems[buf]).wait()

        def s_start(buf, j):
            pltpu.async_copy(bufs[buf], acc_s.at[srcv.at[j]], ssems[buf],
                             add=True)
            if with_cnt:
                pltpu.async_copy(ones_v, acc_c.at[srcv.at[j]], csem,
                                 add=True)

        def s_wait(buf, j):
            pltpu.make_async_copy(
                bufs[buf], acc_s.at[srcv.at[j]], ssems[buf]).wait()
            if with_cnt:
                pltpu.make_async_copy(
                    ones_v, acc_c.at[srcv.at[j]], csem).wait()

        @pl.loop(0, NGB)
        def _(g):
            pltpu.sync_copy(dsti.at[pl.ds(sid * NCHUNK + g * GB, GB)], dstv)
            pltpu.sync_copy(srci.at[pl.ds(sid * NCHUNK + g * GB, GB)], srcv)
            hd = NBUF // 2
            for t in range(hd):
                g_start(t, t)

            @pl.loop(0, GB, step=NBUF)
            def _(j):
                for t in range(NBUF):
                    b, bp = t, (t + hd) % NBUF
                    c = j + t
                    g_wait(b, c)
                    s_start(b, c)

                    @pl.when(c >= hd)
                    def _():
                        s_wait(bp, c - hd)

                    @pl.when(c + hd < GB)
                    def _():
                        g_start(bp, c + hd)

            for t in range(hd):
                s_wait(t + hd, GB - hd + t)

    @pl.when(cid == 0)
    def _():
        run(beta0, True)

    @pl.when(cid == 1)
    def _():
        run(beta1, False)

    plsc.subcore_barrier()

    obufs = (rows, rows1, rows2, rows3, rows4)
    osems = (gsa, gsb, gsc, gsd, gse)
    for k in range(RPT // CH):
        base = sid * RPT + k * CH
        pltpu.sync_copy(acc_s.at[pl.ds(base, CH)], obufs[k])

        @pl.when(cid == 0)
        def _():
            pltpu.async_copy(obufs[k], s0_o.at[pl.ds(base, CH)], osems[k])

        @pl.when(cid == 1)
        def _():
            pltpu.async_copy(obufs[k], s1_o.at[pl.ds(base, CH)], osems[k])

    @pl.when(cid == 0)
    def _():
        @pl.loop(0, RPT // CH)
        def _(k):
            base = sid * RPT + k * CH
            pltpu.sync_copy(acc_c.at[pl.ds(base, CH)], ones_v)
            pltpu.sync_copy(ones_v, cnt_o.at[pl.ds(base, CH)])

    for k in range(RPT // CH):
        base = sid * RPT + k * CH

        @pl.when(cid == 0)
        def _():
            pltpu.make_async_copy(
                obufs[k], s0_o.at[pl.ds(base, CH)], osems[k]).wait()

        @pl.when(cid == 1)
        def _():
            pltpu.make_async_copy(
                obufs[k], s1_o.at[pl.ds(base, CH)], osems[k]).wait()


_sc_aggregate = functools.partial(
    pl.kernel,
    out_type=[
        jax.ShapeDtypeStruct((R, DH), jnp.float32),
        jax.ShapeDtypeStruct((R, DH), jnp.float32),
        jax.ShapeDtypeStruct((R, CNTW), jnp.float32),
    ],
    mesh=plsc.VectorSubcoreMesh(core_axis_name="c", subcore_axis_name="s"),
    compiler_params=pltpu.CompilerParams(use_tc_tiling_on_sc=False),
    scratch_types=[
        pltpu.VMEM((GB, CH), jnp.int32),
        pltpu.VMEM((GB, CH), jnp.int32),
    ] + [pltpu.VMEM((CH, DH), jnp.float32)] * NBUF
      + [
        pltpu.VMEM((CH, CNTW), jnp.float32),
        pltpu.VMEM_SHARED((R, DH), jnp.float32),
        pltpu.VMEM_SHARED((R, CNTW), jnp.float32),
    ] + [pltpu.SemaphoreType.DMA] * (2 * NBUF + 1),
)(_sc_body)


def kernel(x, edge_index, wc, wn, bias):
    src = edge_index[0].astype(jnp.int32)
    dst = edge_index[1].astype(jnp.int32)
    pad = EP - E
    pad_iota = jnp.arange(pad, dtype=jnp.int32)
    src_p = jnp.concatenate(
        [src, N + pad_iota % (R - N)]).reshape(EP // CH, CH)
    dst_p = jnp.concatenate(
        [dst, pad_iota % N]).reshape(EP // CH, CH)

    beta0, beta1 = pl.pallas_call(
        _mm_body,
        out_shape=[
            jax.ShapeDtypeStruct((N, DH), jnp.float32),
            jax.ShapeDtypeStruct((N, DH), jnp.float32),
        ],
    )(x, wn)

    s0, s1, cnt = _sc_aggregate(beta0, beta1, dst_p, src_p)

    out = pl.pallas_call(
        _combine_body,
        out_shape=jax.ShapeDtypeStruct((N, D), jnp.float32),
    )(x, wc, bias.reshape(1, D), s0, s1, cnt)
    return out
